# Initial kernel scaffold; baseline (speedup 1.0000x reference)
import jax
import jax.numpy as jnp
from jax.experimental import pallas as pl


def kernel(x, edge_index, W1, b1, W2, b2, W3, b3, W4, b4, W5, b5, W6, b6):
    raise NotImplementedError("write your pallas kernel here")


# speedup vs baseline: 5.7241x; 5.7241x over previous
import functools

import jax
import jax.numpy as jnp
from jax import lax
from jax.experimental import pallas as pl
from jax.experimental.pallas import tpu as pltpu
from jax.experimental.pallas import tpu_sc as plsc

N_NODES = 100000
N_ACC = 100352
DUMMY = N_NODES
E_EDGES = 1600000
K_SUB = 4
OUTER = 196
E_PAD = 16 * K_SUB * 128 * OUTER
IDX_ROWS = E_PAD // 128
TILE_ROWS = K_SUB * OUTER

HALF = 51200
ACC_ROWS = 53248
SLAB = ACC_ROWS // 16
LOC_DUMMY = 52224

ROWS_BLK = 2048
GRID_N = N_ACC // ROWS_BLK


def _remap_dst(c, dstb, dstl, j):
    base = c * HALF
    for v in range(8):
        d16 = dstb[j, pl.ds(v * 16, 16)]
        loc = d16 - base
        ok = (loc >= 0) & (loc < HALF)
        dstl[j, pl.ds(v * 16, 16)] = jnp.where(ok, loc, LOC_DUMMY)


def _edge_body(t_hbm, src_hbm, dst_hbm, out_hbm, srcb, dstb, dstl, rows, zbuf,
               acc, sem):
    c = lax.axis_index("c")
    s = lax.axis_index("s")

    @pl.loop(0, 128)
    def _(i):
        zbuf[i, :] = jnp.zeros((16,), jnp.float32)

    slab0 = s * SLAB

    @pl.loop(0, SLAB // 128)
    def _(j):
        pltpu.sync_copy(zbuf, acc.at[pl.ds(slab0 + j * 128, 128)])

    plsc.subcore_barrier()

    row0 = s * TILE_ROWS

    @pl.loop(0, OUTER)
    def _(o):
        base = row0 + o * K_SUB
        pltpu.sync_copy(src_hbm.at[pl.ds(base, K_SUB)], srcb)
        pltpu.sync_copy(dst_hbm.at[pl.ds(base, K_SUB)], dstb)
        cps = [
            pltpu.async_copy(t_hbm.at[srcb.at[j]], rows.at[j], sem)
            for j in range(K_SUB)
        ]
        for j in range(K_SUB):
            _remap_dst(c, dstb, dstl, j)
        for cp in cps:
            cp.wait()
        for j in range(K_SUB):
            pltpu.sync_copy(rows.at[j], acc.at[dstl.at[j]], add=True)

    plsc.subcore_barrier()
    pltpu.sync_copy(acc.at[pl.ds(slab0, SLAB)],
                    out_hbm.at[c, pl.ds(slab0, SLAB)])


def _degree_body(dst_hbm, out_hbm, dstb, dstl, onesb, zbuf, acc):
    c = lax.axis_index("c")
    s = lax.axis_index("s")

    @pl.loop(0, 128)
    def _(i):
        zbuf[i, :] = jnp.zeros((16,), jnp.float32)
        onesb[i, :] = jnp.ones((16,), jnp.float32)

    slab0 = s * SLAB

    @pl.loop(0, SLAB // 128)
    def _(j):
        pltpu.sync_copy(zbuf, acc.at[pl.ds(slab0 + j * 128, 128)])

    plsc.subcore_barrier()

    row0 = s * TILE_ROWS

    @pl.loop(0, OUTER)
    def _(o):
        base = row0 + o * K_SUB
        pltpu.sync_copy(dst_hbm.at[pl.ds(base, K_SUB)], dstb)
        for j in range(K_SUB):
            _remap_dst(c, dstb, dstl, j)
        for j in range(K_SUB):
            pltpu.sync_copy(onesb, acc.at[dstl.at[j]], add=True)

    plsc.subcore_barrier()
    pltpu.sync_copy(acc.at[pl.ds(slab0, SLAB)],
                    out_hbm.at[c, pl.ds(slab0, SLAB)])


@functools.cache
def _sc_mesh():
    return plsc.VectorSubcoreMesh(core_axis_name="c", subcore_axis_name="s",
                                  num_cores=2, num_subcores=16)


@functools.cache
def _edge_pass_fn():
    return pl.kernel(
        _edge_body,
        mesh=_sc_mesh(),
        compiler_params=pltpu.CompilerParams(use_tc_tiling_on_sc=False),
        out_type=jax.ShapeDtypeStruct((2, ACC_ROWS, 16), jnp.float32),
        scratch_types=[
            pltpu.VMEM((K_SUB, 128), jnp.int32),
            pltpu.VMEM((K_SUB, 128), jnp.int32),
            pltpu.VMEM((K_SUB, 128), jnp.int32),
            pltpu.VMEM((K_SUB, 128, 16), jnp.float32),
            pltpu.VMEM((128, 16), jnp.float32),
            pltpu.VMEM_SHARED((ACC_ROWS, 16), jnp.float32),
            pltpu.SemaphoreType.DMA,
        ],
    )


@functools.cache
def _degree_pass_fn():
    return pl.kernel(
        _degree_body,
        mesh=_sc_mesh(),
        compiler_params=pltpu.CompilerParams(use_tc_tiling_on_sc=False),
        out_type=jax.ShapeDtypeStruct((2, ACC_ROWS, 16), jnp.float32),
        scratch_types=[
            pltpu.VMEM((K_SUB, 128), jnp.int32),
            pltpu.VMEM((K_SUB, 128), jnp.int32),
            pltpu.VMEM((128, 16), jnp.float32),
            pltpu.VMEM((128, 16), jnp.float32),
            pltpu.VMEM_SHARED((ACC_ROWS, 16), jnp.float32),
        ],
    )


def _assemble(two_halves):
    return jnp.concatenate(
        [two_halves[0, :HALF], two_halves[1, :N_ACC - HALF]], axis=0)


def _edge_pass(t, src2d, dst2d):
    return _assemble(_edge_pass_fn()(t, src2d, dst2d))


def _degree_pass(dst2d):
    return _assemble(_degree_pass_fn()(dst2d))


def _dinv_body(deg_ref, dinv_ref):
    dinv_ref[...] = lax.rsqrt(deg_ref[...] + 1.0)


def _dinv_call(deg):
    return pl.pallas_call(
        _dinv_body,
        grid=(GRID_N,),
        in_specs=[pl.BlockSpec((ROWS_BLK, 16), lambda i: (i, 0))],
        out_specs=pl.BlockSpec((ROWS_BLK, 16), lambda i: (i, 0)),
        out_shape=jax.ShapeDtypeStruct((N_ACC, 16), jnp.float32),
    )(deg)


def _p1_body(x_ref, dinv_ref, w_ref, out_ref):
    res = jnp.dot(x_ref[...], w_ref[...], preferred_element_type=jnp.float32)
    out_ref[...] = res * dinv_ref[...]


def _p1_call(x_pad, dinv, w_pad):
    return pl.pallas_call(
        _p1_body,
        grid=(GRID_N,),
        in_specs=[
            pl.BlockSpec((ROWS_BLK, 16), lambda i: (i, 0)),
            pl.BlockSpec((ROWS_BLK, 16), lambda i: (i, 0)),
            pl.BlockSpec((16, 16), lambda i: (0, 0)),
        ],
        out_specs=pl.BlockSpec((ROWS_BLK, 16), lambda i: (i, 0)),
        out_shape=jax.ShapeDtypeStruct((N_ACC, 16), jnp.float32),
    )(x_pad, dinv, w_pad)


def _make_p_body(nc_in, nc_out, act):
    def body(*refs):
        t_refs = refs[0:nc_in]
        p_refs = refs[nc_in:2 * nc_in]
        dinv_ref = refs[2 * nc_in]
        b_ref = refs[2 * nc_in + 1]
        w_ref = refs[2 * nc_in + 2]
        out_refs = refs[2 * nc_in + 3:]
        dinv = dinv_ref[...]
        hs = []
        for ci in range(nc_in):
            agg = (p_refs[ci][...] + t_refs[ci][...]) * dinv
            y = agg + b_ref[0, ci * 16:(ci + 1) * 16]
            hs.append(jnp.maximum(y, 0.0) if act else y)
        h = jnp.concatenate(hs, axis=1) if nc_in > 1 else hs[0]
        res = jnp.dot(h, w_ref[...], preferred_element_type=jnp.float32)
        t = res * dinv[:, 0:1]
        for cj in range(nc_out):
            out_refs[cj][...] = t[:, cj * 16:(cj + 1) * 16]
    return body


def _p_call(nc_in, nc_out, act, t_chunks, p_chunks, dinv, b_pad, w_pad):
    in_specs = (
        [pl.BlockSpec((ROWS_BLK, 16), lambda i: (i, 0))] * (2 * nc_in)
        + [pl.BlockSpec((ROWS_BLK, 16), lambda i: (i, 0)),
           pl.BlockSpec((1, nc_in * 16), lambda i: (0, 0)),
           pl.BlockSpec((nc_in * 16, nc_out * 16), lambda i: (0, 0))]
    )
    out_specs = [pl.BlockSpec((ROWS_BLK, 16), lambda i: (i, 0))] * nc_out
    out_shape = [jax.ShapeDtypeStruct((N_ACC, 16), jnp.float32)] * nc_out
    res = pl.pallas_call(
        _make_p_body(nc_in, nc_out, act),
        grid=(GRID_N,),
        in_specs=in_specs,
        out_specs=out_specs,
        out_shape=out_shape,
    )(*t_chunks, *p_chunks, dinv, b_pad, w_pad)
    return list(res)


def _final_body(t_ref, p_ref, dinv_ref, b_ref, out_ref):
    y = (p_ref[...] + t_ref[...]) * dinv_ref[...] + b_ref[0]
    out_ref[...] = y


def _final_call(t6, p6, dinv, b_pad):
    return pl.pallas_call(
        _final_body,
        grid=(GRID_N,),
        in_specs=[
            pl.BlockSpec((ROWS_BLK, 16), lambda i: (i, 0)),
            pl.BlockSpec((ROWS_BLK, 16), lambda i: (i, 0)),
            pl.BlockSpec((ROWS_BLK, 16), lambda i: (i, 0)),
            pl.BlockSpec((1, 16), lambda i: (0, 0)),
        ],
        out_specs=pl.BlockSpec((ROWS_BLK, 16), lambda i: (i, 0)),
        out_shape=jax.ShapeDtypeStruct((N_ACC, 16), jnp.float32),
    )(t6, p6, dinv, b_pad)


def _pad_w(w, rows, cols):
    return jnp.pad(w, ((0, rows - w.shape[0]), (0, cols - w.shape[1])))


def _pad_b(b, cols):
    return jnp.pad(b, (0, cols - b.shape[0])).reshape(1, cols)


def kernel(x, edge_index, W1, b1, W2, b2, W3, b3, W4, b4, W5, b5, W6, b6):
    n, _ = x.shape
    e = edge_index.shape[1]
    pad_e = E_PAD - e
    src2d = jnp.concatenate(
        [edge_index[0], jnp.full((pad_e,), DUMMY, jnp.int32)]).reshape(
            IDX_ROWS, 128)
    dst2d = jnp.concatenate(
        [edge_index[1], jnp.full((pad_e,), DUMMY, jnp.int32)]).reshape(
            IDX_ROWS, 128)

    x_pad = jnp.pad(x, ((0, N_ACC - n), (0, 16 - x.shape[1])))

    deg = _degree_pass(dst2d)
    dinv = _dinv_call(deg)

    dims = [11, 16, 32, 50, 32, 16, 11]
    ncs = [-(-d // 16) for d in dims]
    ws = [W1, W2, W3, W4, W5, W6]
    bs = [b1, b2, b3, b4, b5, b6]
    acts = [True, True, False, True, True, False]

    t_chunks = [_p1_call(x_pad, dinv, _pad_w(W1, 16, ncs[1] * 16))]

    out_full = None
    for li in range(6):
        nc_in = ncs[li + 1]
        p_chunks = [_edge_pass(t_chunks[ci], src2d, dst2d)
                    for ci in range(nc_in)]
        if li < 5:
            nc_out = ncs[li + 2]
            t_chunks = _p_call(
                nc_in, nc_out, acts[li], t_chunks, p_chunks, dinv,
                _pad_b(bs[li], nc_in * 16),
                _pad_w(ws[li + 1], nc_in * 16, nc_out * 16))
        else:
            out_full = _final_call(t_chunks[0], p_chunks[0], dinv,
                                   _pad_b(bs[5], 16))
    return out_full[:n, :dims[6]]


# speedup vs baseline: 9.1872x; 1.6050x over previous
import functools

import jax
import jax.numpy as jnp
from jax import lax
from jax.experimental import pallas as pl
from jax.experimental.pallas import tpu as pltpu
from jax.experimental.pallas import tpu_sc as plsc

N_NODES = 100000
N_ACC = 100352
E_EDGES = 1600000
K_SUB = 4
OUTER = 196
E_PAD = 16 * K_SUB * 128 * OUTER
IDX_ROWS = E_PAD // 128
TILE_ROWS = K_SUB * OUTER

HALF = 51200
ACC_ROWS = 51200
SLAB = ACC_ROWS // 16
SKIP = -1

ROWS_BLK = 2048
GRID_N = N_ACC // ROWS_BLK


def _remap_dst(c, dstb, dstl, j):
    base = c * HALF
    for v in range(8):
        d16 = dstb[j, pl.ds(v * 16, 16)]
        loc = d16 - base
        ok = (loc >= 0) & (loc < HALF)
        dstl[j, pl.ds(v * 16, 16)] = jnp.where(ok, loc, SKIP)


def _remap_both(c, srcb, dstb, srcm, dstl, j):
    base = c * HALF
    for v in range(8):
        sl = pl.ds(v * 16, 16)
        d16 = dstb[j, sl]
        loc = d16 - base
        ok = (loc >= 0) & (loc < HALF)
        dstl[j, sl] = jnp.where(ok, loc, SKIP)
        srcm[j, sl] = jnp.where(ok, srcb[j, sl], SKIP)


def _edge_body(t_hbm, src_hbm, dst_hbm, out_hbm, srcb, dstb, srcm, dstl, rows,
               zbuf, acc, sem):
    c = lax.axis_index("c")
    s = lax.axis_index("s")

    @pl.loop(0, 128)
    def _(i):
        zbuf[i, :] = jnp.zeros((16,), jnp.float32)

    slab0 = s * SLAB

    @pl.loop(0, SLAB // 128)
    def _(j):
        pltpu.sync_copy(zbuf, acc.at[pl.ds(slab0 + j * 128, 128)])

    plsc.subcore_barrier()

    row0 = s * TILE_ROWS

    @pl.loop(0, OUTER)
    def _(o):
        base = row0 + o * K_SUB
        pltpu.sync_copy(src_hbm.at[pl.ds(base, K_SUB)], srcb)
        pltpu.sync_copy(dst_hbm.at[pl.ds(base, K_SUB)], dstb)
        for j in range(K_SUB):
            _remap_both(c, srcb, dstb, srcm, dstl, j)
        cps = [
            pltpu.async_copy(
                t_hbm.at[plsc.Indices(srcm.at[j], ignored_value=SKIP)],
                rows.at[j], sem)
            for j in range(K_SUB)
        ]
        for cp in cps:
            cp.wait()
        for j in range(K_SUB):
            pltpu.sync_copy(
                rows.at[j],
                acc.at[plsc.Indices(dstl.at[j], ignored_value=SKIP)],
                add=True)

    plsc.subcore_barrier()
    pltpu.sync_copy(acc.at[pl.ds(slab0, SLAB)],
                    out_hbm.at[c, pl.ds(slab0, SLAB)])


def _degree_body(dst_hbm, out_hbm, dstb, dstl, onesb, zbuf, acc):
    c = lax.axis_index("c")
    s = lax.axis_index("s")

    @pl.loop(0, 128)
    def _(i):
        zbuf[i, :] = jnp.zeros((16,), jnp.float32)
        onesb[i, :] = jnp.ones((16,), jnp.float32)

    slab0 = s * SLAB

    @pl.loop(0, SLAB // 128)
    def _(j):
        pltpu.sync_copy(zbuf, acc.at[pl.ds(slab0 + j * 128, 128)])

    plsc.subcore_barrier()

    row0 = s * TILE_ROWS

    @pl.loop(0, OUTER)
    def _(o):
        base = row0 + o * K_SUB
        pltpu.sync_copy(dst_hbm.at[pl.ds(base, K_SUB)], dstb)
        for j in range(K_SUB):
            _remap_dst(c, dstb, dstl, j)
        for j in range(K_SUB):
            pltpu.sync_copy(
                onesb, acc.at[plsc.Indices(dstl.at[j], ignored_value=SKIP)],
                add=True)

    plsc.subcore_barrier()
    pltpu.sync_copy(acc.at[pl.ds(slab0, SLAB)],
                    out_hbm.at[c, pl.ds(slab0, SLAB)])


@functools.cache
def _sc_mesh():
    return plsc.VectorSubcoreMesh(core_axis_name="c", subcore_axis_name="s",
                                  num_cores=2, num_subcores=16)


@functools.cache
def _edge_pass_fn():
    return pl.kernel(
        _edge_body,
        mesh=_sc_mesh(),
        compiler_params=pltpu.CompilerParams(use_tc_tiling_on_sc=False),
        out_type=jax.ShapeDtypeStruct((2, ACC_ROWS, 16), jnp.float32),
        scratch_types=[
            pltpu.VMEM((K_SUB, 128), jnp.int32),
            pltpu.VMEM((K_SUB, 128), jnp.int32),
            pltpu.VMEM((K_SUB, 128), jnp.int32),
            pltpu.VMEM((K_SUB, 128), jnp.int32),
            pltpu.VMEM((K_SUB, 128, 16), jnp.float32),
            pltpu.VMEM((128, 16), jnp.float32),
            pltpu.VMEM_SHARED((ACC_ROWS, 16), jnp.float32),
            pltpu.SemaphoreType.DMA,
        ],
    )


@functools.cache
def _degree_pass_fn():
    return pl.kernel(
        _degree_body,
        mesh=_sc_mesh(),
        compiler_params=pltpu.CompilerParams(use_tc_tiling_on_sc=False),
        out_type=jax.ShapeDtypeStruct((2, ACC_ROWS, 16), jnp.float32),
        scratch_types=[
            pltpu.VMEM((K_SUB, 128), jnp.int32),
            pltpu.VMEM((K_SUB, 128), jnp.int32),
            pltpu.VMEM((128, 16), jnp.float32),
            pltpu.VMEM((128, 16), jnp.float32),
            pltpu.VMEM_SHARED((ACC_ROWS, 16), jnp.float32),
        ],
    )


def _assemble(two_halves):
    return jnp.concatenate(
        [two_halves[0, :HALF], two_halves[1, :N_ACC - HALF]], axis=0)


def _edge_pass(t, src2d, dst2d):
    return _assemble(_edge_pass_fn()(t, src2d, dst2d))


def _degree_pass(dst2d):
    return _assemble(_degree_pass_fn()(dst2d))


def _dinv_body(deg_ref, dinv_ref):
    dinv_ref[...] = lax.rsqrt(deg_ref[...] + 1.0)


def _dinv_call(deg):
    return pl.pallas_call(
        _dinv_body,
        grid=(GRID_N,),
        in_specs=[pl.BlockSpec((ROWS_BLK, 16), lambda i: (i, 0))],
        out_specs=pl.BlockSpec((ROWS_BLK, 16), lambda i: (i, 0)),
        out_shape=jax.ShapeDtypeStruct((N_ACC, 16), jnp.float32),
    )(deg)


def _p1_body(x_ref, dinv_ref, w_ref, out_ref):
    res = jnp.dot(x_ref[...], w_ref[...], preferred_element_type=jnp.float32)
    out_ref[...] = res * dinv_ref[...]


def _p1_call(x_pad, dinv, w_pad):
    return pl.pallas_call(
        _p1_body,
        grid=(GRID_N,),
        in_specs=[
            pl.BlockSpec((ROWS_BLK, 16), lambda i: (i, 0)),
            pl.BlockSpec((ROWS_BLK, 16), lambda i: (i, 0)),
            pl.BlockSpec((16, 16), lambda i: (0, 0)),
        ],
        out_specs=pl.BlockSpec((ROWS_BLK, 16), lambda i: (i, 0)),
        out_shape=jax.ShapeDtypeStruct((N_ACC, 16), jnp.float32),
    )(x_pad, dinv, w_pad)


def _make_p_body(nc_in, nc_out, act):
    def body(*refs):
        t_refs = refs[0:nc_in]
        p_refs = refs[nc_in:2 * nc_in]
        dinv_ref = refs[2 * nc_in]
        b_ref = refs[2 * nc_in + 1]
        w_ref = refs[2 * nc_in + 2]
        out_refs = refs[2 * nc_in + 3:]
        dinv = dinv_ref[...]
        hs = []
        for ci in range(nc_in):
            agg = (p_refs[ci][...] + t_refs[ci][...]) * dinv
            y = agg + b_ref[0, ci * 16:(ci + 1) * 16]
            hs.append(jnp.maximum(y, 0.0) if act else y)
        h = jnp.concatenate(hs, axis=1) if nc_in > 1 else hs[0]
        res = jnp.dot(h, w_ref[...], preferred_element_type=jnp.float32)
        t = res * dinv[:, 0:1]
        for cj in range(nc_out):
            out_refs[cj][...] = t[:, cj * 16:(cj + 1) * 16]
    return body


def _p_call(nc_in, nc_out, act, t_chunks, p_chunks, dinv, b_pad, w_pad):
    in_specs = (
        [pl.BlockSpec((ROWS_BLK, 16), lambda i: (i, 0))] * (2 * nc_in)
        + [pl.BlockSpec((ROWS_BLK, 16), lambda i: (i, 0)),
           pl.BlockSpec((1, nc_in * 16), lambda i: (0, 0)),
           pl.BlockSpec((nc_in * 16, nc_out * 16), lambda i: (0, 0))]
    )
    out_specs = [pl.BlockSpec((ROWS_BLK, 16), lambda i: (i, 0))] * nc_out
    out_shape = [jax.ShapeDtypeStruct((N_ACC, 16), jnp.float32)] * nc_out
    res = pl.pallas_call(
        _make_p_body(nc_in, nc_out, act),
        grid=(GRID_N,),
        in_specs=in_specs,
        out_specs=out_specs,
        out_shape=out_shape,
    )(*t_chunks, *p_chunks, dinv, b_pad, w_pad)
    return list(res)


def _final_body(t_ref, p_ref, dinv_ref, b_ref, out_ref):
    y = (p_ref[...] + t_ref[...]) * dinv_ref[...] + b_ref[0]
    out_ref[...] = y


def _final_call(t6, p6, dinv, b_pad):
    return pl.pallas_call(
        _final_body,
        grid=(GRID_N,),
        in_specs=[
            pl.BlockSpec((ROWS_BLK, 16), lambda i: (i, 0)),
            pl.BlockSpec((ROWS_BLK, 16), lambda i: (i, 0)),
            pl.BlockSpec((ROWS_BLK, 16), lambda i: (i, 0)),
            pl.BlockSpec((1, 16), lambda i: (0, 0)),
        ],
        out_specs=pl.BlockSpec((ROWS_BLK, 16), lambda i: (i, 0)),
        out_shape=jax.ShapeDtypeStruct((N_ACC, 16), jnp.float32),
    )(t6, p6, dinv, b_pad)


def _pad_w(w, rows, cols):
    return jnp.pad(w, ((0, rows - w.shape[0]), (0, cols - w.shape[1])))


def _pad_b(b, cols):
    return jnp.pad(b, (0, cols - b.shape[0])).reshape(1, cols)


def kernel(x, edge_index, W1, b1, W2, b2, W3, b3, W4, b4, W5, b5, W6, b6):
    n, _ = x.shape
    e = edge_index.shape[1]
    pad_e = E_PAD - e
    src2d = jnp.concatenate(
        [edge_index[0], jnp.zeros((pad_e,), jnp.int32)]).reshape(
            IDX_ROWS, 128)
    dst2d = jnp.concatenate(
        [edge_index[1], jnp.full((pad_e,), SKIP, jnp.int32)]).reshape(
            IDX_ROWS, 128)

    x_pad = jnp.pad(x, ((0, N_ACC - n), (0, 16 - x.shape[1])))

    deg = _degree_pass(dst2d)
    dinv = _dinv_call(deg)

    dims = [11, 16, 32, 50, 32, 16, 11]
    ncs = [-(-d // 16) for d in dims]
    ws = [W1, W2, W3, W4, W5, W6]
    bs = [b1, b2, b3, b4, b5, b6]
    acts = [True, True, False, True, True, False]

    t_chunks = [_p1_call(x_pad, dinv, _pad_w(W1, 16, ncs[1] * 16))]

    out_full = None
    for li in range(6):
        nc_in = ncs[li + 1]
        p_chunks = [_edge_pass(t_chunks[ci], src2d, dst2d)
                    for ci in range(nc_in)]
        if li < 5:
            nc_out = ncs[li + 2]
            t_chunks = _p_call(
                nc_in, nc_out, acts[li], t_chunks, p_chunks, dinv,
                _pad_b(bs[li], nc_in * 16),
                _pad_w(ws[li + 1], nc_in * 16, nc_out * 16))
        else:
            out_full = _final_call(t_chunks[0], p_chunks[0], dinv,
                                   _pad_b(bs[5], 16))
    return out_full[:n, :dims[6]]


# speedup vs baseline: 20.8646x; 2.2710x over previous
import functools

import jax
import jax.numpy as jnp
from jax import lax
from jax.experimental import pallas as pl
from jax.experimental.pallas import tpu as pltpu
from jax.experimental.pallas import tpu_sc as plsc

N_NODES = 100000
N_ACC = 100352
E_EDGES = 1600000
K_SUB = 4
OUTER = 196
E_PAD = 16 * K_SUB * 128 * OUTER
IDX_ROWS = E_PAD // 128
TILE_ROWS = K_SUB * OUTER

HALF = 51200
ACC_ROWS = 51200
SLAB = ACC_ROWS // 16
SKIP = -1

ROWS_BLK = 2048
GRID_N = N_ACC // ROWS_BLK


DEPTH = 4
CHUNKS = OUTER


def _remap_dst(c, dstb, dstl, p, j):
    base = c * HALF
    for v in range(8):
        sl = pl.ds(v * 16, 16)
        loc = dstb[p, j, sl] - base
        ok = (loc >= 0) & (loc < HALF)
        dstl[p, j, sl] = jnp.where(ok, loc, SKIP)


def _remap_both(c, srcb, dstb, srcm, dstl, p, j):
    base = c * HALF
    for v in range(8):
        sl = pl.ds(v * 16, 16)
        loc = dstb[p, j, sl] - base
        ok = (loc >= 0) & (loc < HALF)
        dstl[p, j, sl] = jnp.where(ok, loc, SKIP)
        srcm[p, j, sl] = jnp.where(ok, srcb[p, j, sl], SKIP)


def _zero_slab(s, zbuf, acc, onesb=None):
    @pl.loop(0, 128)
    def _(i):
        zbuf[i, :] = jnp.zeros((16,), jnp.float32)
        if onesb is not None:
            onesb[i, :] = jnp.ones((16,), jnp.float32)

    slab0 = s * SLAB

    @pl.loop(0, SLAB // 128)
    def _(j):
        pltpu.sync_copy(zbuf, acc.at[pl.ds(slab0 + j * 128, 128)])

    return slab0


def _idx_fire(hbm, buf, sem, slot, base):
    pltpu.async_copy(hbm.at[pl.ds(base, K_SUB)], buf.at[slot], sem.at[slot])


def _idx_drain(hbm, buf, sem, slot):
    pltpu.make_async_copy(hbm.at[pl.ds(0, K_SUB)], buf.at[slot],
                          sem.at[slot]).wait()


def _gather_fire(t_hbm, srcm, rows, gsem, p):
    for j in range(K_SUB):
        pltpu.async_copy(
            t_hbm.at[plsc.Indices(srcm.at[p, j], ignored_value=SKIP)],
            rows.at[p, j], gsem.at[p])


def _gather_drain(t_hbm, srcm, rows, gsem, p):
    for j in range(K_SUB):
        pltpu.make_async_copy(
            t_hbm.at[plsc.Indices(srcm.at[p, j], ignored_value=SKIP)],
            rows.at[p, j], gsem.at[p]).wait()


def _scatter_fire(rows, acc, dstl, ssem, p):
    for j in range(K_SUB):
        pltpu.async_copy(
            rows.at[p, j],
            acc.at[plsc.Indices(dstl.at[p, j], ignored_value=SKIP)],
            ssem.at[p], add=True)


def _scatter_drain(rows, acc, dstl, ssem, p):
    for j in range(K_SUB):
        pltpu.make_async_copy(
            rows.at[p, j],
            acc.at[plsc.Indices(dstl.at[p, j], ignored_value=SKIP)],
            ssem.at[p]).wait()


def _edge_body(t_hbm, src_hbm, dst_hbm, out_hbm, srcb, dstb, srcm, dstl, rows,
               zbuf, acc, isem, gsem, ssem):
    c = lax.axis_index("c")
    s = lax.axis_index("s")
    slab0 = _zero_slab(s, zbuf, acc)
    plsc.subcore_barrier()

    row0 = s * TILE_ROWS
    for k0 in range(3):
        _idx_fire(src_hbm, srcb, isem, k0, row0 + k0 * K_SUB)
        _idx_fire(dst_hbm, dstb, isem, k0, row0 + k0 * K_SUB)

    @pl.loop(0, CHUNKS)
    def _(k):
        p = k & 3
        _idx_drain(src_hbm, srcb, isem, p)
        _idx_drain(dst_hbm, dstb, isem, p)

        @pl.when(k >= 4)
        def _():
            _scatter_drain(rows, acc, dstl, ssem, p)

        for j in range(K_SUB):
            _remap_both(c, srcb, dstb, srcm, dstl, p, j)
        _gather_fire(t_hbm, srcm, rows, gsem, p)

        @pl.when(k + 3 < CHUNKS)
        def _():
            base3 = row0 + (k + 3) * K_SUB
            pp3 = (k + 3) & 3
            _idx_fire(src_hbm, srcb, isem, pp3, base3)
            _idx_fire(dst_hbm, dstb, isem, pp3, base3)

        @pl.when(k >= 1)
        def _():
            pm1 = (k - 1) & 3
            _gather_drain(t_hbm, srcm, rows, gsem, pm1)
            _scatter_fire(rows, acc, dstl, ssem, pm1)

    plast = (CHUNKS - 1) & 3
    _gather_drain(t_hbm, srcm, rows, gsem, plast)
    _scatter_fire(rows, acc, dstl, ssem, plast)
    for q in range(4):
        _scatter_drain(rows, acc, dstl, ssem, q)

    plsc.subcore_barrier()
    pltpu.sync_copy(acc.at[pl.ds(slab0, SLAB)],
                    out_hbm.at[c, pl.ds(slab0, SLAB)])


def _ones_scatter_fire(onesb, acc, dstl, ssem, p):
    for j in range(K_SUB):
        pltpu.async_copy(
            onesb, acc.at[plsc.Indices(dstl.at[p, j], ignored_value=SKIP)],
            ssem.at[p], add=True)


def _ones_scatter_drain(onesb, acc, dstl, ssem, p):
    for j in range(K_SUB):
        pltpu.make_async_copy(
            onesb, acc.at[plsc.Indices(dstl.at[p, j], ignored_value=SKIP)],
            ssem.at[p]).wait()


def _degree_body(dst_hbm, out_hbm, dstb, dstl, onesb, zbuf, acc, isem, ssem):
    c = lax.axis_index("c")
    s = lax.axis_index("s")
    slab0 = _zero_slab(s, zbuf, acc, onesb=onesb)
    plsc.subcore_barrier()

    row0 = s * TILE_ROWS
    for k0 in range(3):
        _idx_fire(dst_hbm, dstb, isem, k0, row0 + k0 * K_SUB)

    @pl.loop(0, CHUNKS)
    def _(k):
        p = k & 3
        _idx_drain(dst_hbm, dstb, isem, p)

        @pl.when(k >= 4)
        def _():
            _ones_scatter_drain(onesb, acc, dstl, ssem, p)

        for j in range(K_SUB):
            _remap_dst(c, dstb, dstl, p, j)
        _ones_scatter_fire(onesb, acc, dstl, ssem, p)

        @pl.when(k + 3 < CHUNKS)
        def _():
            _idx_fire(dst_hbm, dstb, isem, (k + 3) & 3, row0 + (k + 3) * K_SUB)

    for q in range(4):
        _ones_scatter_drain(onesb, acc, dstl, ssem, q)

    plsc.subcore_barrier()
    pltpu.sync_copy(acc.at[pl.ds(slab0, SLAB)],
                    out_hbm.at[c, pl.ds(slab0, SLAB)])


@functools.cache
def _sc_mesh():
    return plsc.VectorSubcoreMesh(core_axis_name="c", subcore_axis_name="s",
                                  num_cores=2, num_subcores=16)


@functools.cache
def _edge_pass_fn():
    return pl.kernel(
        _edge_body,
        mesh=_sc_mesh(),
        compiler_params=pltpu.CompilerParams(use_tc_tiling_on_sc=False),
        out_type=jax.ShapeDtypeStruct((2, ACC_ROWS, 16), jnp.float32),
        scratch_types=[
            pltpu.VMEM((DEPTH, K_SUB, 128), jnp.int32),
            pltpu.VMEM((DEPTH, K_SUB, 128), jnp.int32),
            pltpu.VMEM((DEPTH, K_SUB, 128), jnp.int32),
            pltpu.VMEM((DEPTH, K_SUB, 128), jnp.int32),
            pltpu.VMEM((DEPTH, K_SUB, 128, 16), jnp.float32),
            pltpu.VMEM((128, 16), jnp.float32),
            pltpu.VMEM_SHARED((ACC_ROWS, 16), jnp.float32),
            pltpu.SemaphoreType.DMA((DEPTH,)),
            pltpu.SemaphoreType.DMA((DEPTH,)),
            pltpu.SemaphoreType.DMA((DEPTH,)),
        ],
    )


@functools.cache
def _degree_pass_fn():
    return pl.kernel(
        _degree_body,
        mesh=_sc_mesh(),
        compiler_params=pltpu.CompilerParams(use_tc_tiling_on_sc=False),
        out_type=jax.ShapeDtypeStruct((2, ACC_ROWS, 16), jnp.float32),
        scratch_types=[
            pltpu.VMEM((DEPTH, K_SUB, 128), jnp.int32),
            pltpu.VMEM((DEPTH, K_SUB, 128), jnp.int32),
            pltpu.VMEM((128, 16), jnp.float32),
            pltpu.VMEM((128, 16), jnp.float32),
            pltpu.VMEM_SHARED((ACC_ROWS, 16), jnp.float32),
            pltpu.SemaphoreType.DMA((DEPTH,)),
            pltpu.SemaphoreType.DMA((DEPTH,)),
        ],
    )


def _assemble(two_halves):
    return jnp.concatenate(
        [two_halves[0, :HALF], two_halves[1, :N_ACC - HALF]], axis=0)


def _edge_pass(t, src2d, dst2d):
    return _assemble(_edge_pass_fn()(t, src2d, dst2d))


def _degree_pass(dst2d):
    return _assemble(_degree_pass_fn()(dst2d))


def _dinv_body(deg_ref, dinv_ref):
    dinv_ref[...] = lax.rsqrt(deg_ref[...] + 1.0)


def _dinv_call(deg):
    return pl.pallas_call(
        _dinv_body,
        grid=(GRID_N,),
        in_specs=[pl.BlockSpec((ROWS_BLK, 16), lambda i: (i, 0))],
        out_specs=pl.BlockSpec((ROWS_BLK, 16), lambda i: (i, 0)),
        out_shape=jax.ShapeDtypeStruct((N_ACC, 16), jnp.float32),
    )(deg)


def _p1_body(x_ref, dinv_ref, w_ref, out_ref):
    res = jnp.dot(x_ref[...], w_ref[...], preferred_element_type=jnp.float32)
    out_ref[...] = res * dinv_ref[...]


def _p1_call(x_pad, dinv, w_pad):
    return pl.pallas_call(
        _p1_body,
        grid=(GRID_N,),
        in_specs=[
            pl.BlockSpec((ROWS_BLK, 16), lambda i: (i, 0)),
            pl.BlockSpec((ROWS_BLK, 16), lambda i: (i, 0)),
            pl.BlockSpec((16, 16), lambda i: (0, 0)),
        ],
        out_specs=pl.BlockSpec((ROWS_BLK, 16), lambda i: (i, 0)),
        out_shape=jax.ShapeDtypeStruct((N_ACC, 16), jnp.float32),
    )(x_pad, dinv, w_pad)


def _make_p_body(nc_in, nc_out, act):
    def body(*refs):
        t_refs = refs[0:nc_in]
        p_refs = refs[nc_in:2 * nc_in]
        dinv_ref = refs[2 * nc_in]
        b_ref = refs[2 * nc_in + 1]
        w_ref = refs[2 * nc_in + 2]
        out_refs = refs[2 * nc_in + 3:]
        dinv = dinv_ref[...]
        hs = []
        for ci in range(nc_in):
            agg = (p_refs[ci][...] + t_refs[ci][...]) * dinv
            y = agg + b_ref[0, ci * 16:(ci + 1) * 16]
            hs.append(jnp.maximum(y, 0.0) if act else y)
        h = jnp.concatenate(hs, axis=1) if nc_in > 1 else hs[0]
        res = jnp.dot(h, w_ref[...], preferred_element_type=jnp.float32)
        t = res * dinv[:, 0:1]
        for cj in range(nc_out):
            out_refs[cj][...] = t[:, cj * 16:(cj + 1) * 16]
    return body


def _p_call(nc_in, nc_out, act, t_chunks, p_chunks, dinv, b_pad, w_pad):
    in_specs = (
        [pl.BlockSpec((ROWS_BLK, 16), lambda i: (i, 0))] * (2 * nc_in)
        + [pl.BlockSpec((ROWS_BLK, 16), lambda i: (i, 0)),
           pl.BlockSpec((1, nc_in * 16), lambda i: (0, 0)),
           pl.BlockSpec((nc_in * 16, nc_out * 16), lambda i: (0, 0))]
    )
    out_specs = [pl.BlockSpec((ROWS_BLK, 16), lambda i: (i, 0))] * nc_out
    out_shape = [jax.ShapeDtypeStruct((N_ACC, 16), jnp.float32)] * nc_out
    res = pl.pallas_call(
        _make_p_body(nc_in, nc_out, act),
        grid=(GRID_N,),
        in_specs=in_specs,
        out_specs=out_specs,
        out_shape=out_shape,
    )(*t_chunks, *p_chunks, dinv, b_pad, w_pad)
    return list(res)


def _final_body(t_ref, p_ref, dinv_ref, b_ref, out_ref):
    y = (p_ref[...] + t_ref[...]) * dinv_ref[...] + b_ref[0]
    out_ref[...] = y


def _final_call(t6, p6, dinv, b_pad):
    return pl.pallas_call(
        _final_body,
        grid=(GRID_N,),
        in_specs=[
            pl.BlockSpec((ROWS_BLK, 16), lambda i: (i, 0)),
            pl.BlockSpec((ROWS_BLK, 16), lambda i: (i, 0)),
            pl.BlockSpec((ROWS_BLK, 16), lambda i: (i, 0)),
            pl.BlockSpec((1, 16), lambda i: (0, 0)),
        ],
        out_specs=pl.BlockSpec((ROWS_BLK, 16), lambda i: (i, 0)),
        out_shape=jax.ShapeDtypeStruct((N_ACC, 16), jnp.float32),
    )(t6, p6, dinv, b_pad)


def _pad_w(w, rows, cols):
    return jnp.pad(w, ((0, rows - w.shape[0]), (0, cols - w.shape[1])))


def _pad_b(b, cols):
    return jnp.pad(b, (0, cols - b.shape[0])).reshape(1, cols)


def kernel(x, edge_index, W1, b1, W2, b2, W3, b3, W4, b4, W5, b5, W6, b6):
    n, _ = x.shape
    e = edge_index.shape[1]
    pad_e = E_PAD - e
    src2d = jnp.concatenate(
        [edge_index[0], jnp.zeros((pad_e,), jnp.int32)]).reshape(
            IDX_ROWS, 128)
    dst2d = jnp.concatenate(
        [edge_index[1], jnp.full((pad_e,), SKIP, jnp.int32)]).reshape(
            IDX_ROWS, 128)

    x_pad = jnp.pad(x, ((0, N_ACC - n), (0, 16 - x.shape[1])))

    deg = _degree_pass(dst2d)
    dinv = _dinv_call(deg)

    dims = [11, 16, 32, 50, 32, 16, 11]
    ncs = [-(-d // 16) for d in dims]
    ws = [W1, W2, W3, W4, W5, W6]
    bs = [b1, b2, b3, b4, b5, b6]
    acts = [True, True, False, True, True, False]

    t_chunks = [_p1_call(x_pad, dinv, _pad_w(W1, 16, ncs[1] * 16))]

    out_full = None
    for li in range(6):
        nc_in = ncs[li + 1]
        p_chunks = [_edge_pass(t_chunks[ci], src2d, dst2d)
                    for ci in range(nc_in)]
        if li < 5:
            nc_out = ncs[li + 2]
            t_chunks = _p_call(
                nc_in, nc_out, acts[li], t_chunks, p_chunks, dinv,
                _pad_b(bs[li], nc_in * 16),
                _pad_w(ws[li + 1], nc_in * 16, nc_out * 16))
        else:
            out_full = _final_call(t_chunks[0], p_chunks[0], dinv,
                                   _pad_b(bs[5], 16))
    return out_full[:n, :dims[6]]


# speedup vs baseline: 24.1090x; 1.1555x over previous
import functools

import jax
import jax.numpy as jnp
from jax import lax
from jax.experimental import pallas as pl
from jax.experimental.pallas import tpu as pltpu
from jax.experimental.pallas import tpu_sc as plsc

N_NODES = 100000
N_ACC = 100352
E_EDGES = 1600000
K_SUB = 4
OUTER = 196
E_PAD = 16 * K_SUB * 128 * OUTER
IDX_ROWS = E_PAD // 128
TILE_ROWS = K_SUB * OUTER

HALF = 51200
ACC_ROWS = 51200
SLAB = ACC_ROWS // 16
SKIP = -1

ROWS_BLK = 2048
GRID_N = N_ACC // ROWS_BLK


DEPTH = 4
CHUNKS = OUTER


def _remap_dst(c, dstb, dstl, p, j):
    base = c * HALF
    for v in range(8):
        sl = pl.ds(v * 16, 16)
        loc = dstb[p, j, sl] - base
        ok = (loc >= 0) & (loc < HALF)
        dstl[p, j, sl] = jnp.where(ok, loc, SKIP)


def _remap_both(c, srcb, dstb, srcm, dstl, p, j):
    base = c * HALF
    for v in range(8):
        sl = pl.ds(v * 16, 16)
        loc = dstb[p, j, sl] - base
        ok = (loc >= 0) & (loc < HALF)
        dstl[p, j, sl] = jnp.where(ok, loc, SKIP)
        srcm[p, j, sl] = jnp.where(ok, srcb[p, j, sl], SKIP)


def _zero_slab(s, zbuf, acc, onesb=None):
    @pl.loop(0, 128)
    def _(i):
        zbuf[i, :] = jnp.zeros((16,), jnp.float32)
        if onesb is not None:
            onesb[i, :] = jnp.ones((16,), jnp.float32)

    slab0 = s * SLAB

    @pl.loop(0, SLAB // 128)
    def _(j):
        pltpu.sync_copy(zbuf, acc.at[pl.ds(slab0 + j * 128, 128)])

    return slab0


def _idx_fire(hbm, buf, sem, slot, base):
    pltpu.async_copy(hbm.at[pl.ds(base, K_SUB)], buf.at[slot], sem.at[slot])


def _idx_drain(hbm, buf, sem, slot):
    pltpu.make_async_copy(hbm.at[pl.ds(0, K_SUB)], buf.at[slot],
                          sem.at[slot]).wait()


def _gather_fire(t_hbm, srcm, rows, gsem, p):
    for j in range(K_SUB):
        pltpu.async_copy(
            t_hbm.at[plsc.Indices(srcm.at[p, j], ignored_value=SKIP)],
            rows.at[p, j], gsem.at[p])


def _gather_drain(t_hbm, srcm, rows, gsem, p):
    for j in range(K_SUB):
        pltpu.make_async_copy(
            t_hbm.at[plsc.Indices(srcm.at[p, j], ignored_value=SKIP)],
            rows.at[p, j], gsem.at[p]).wait()


def _scatter_fire(rows, acc, dstl, ssem, p):
    for j in range(K_SUB):
        pltpu.async_copy(
            rows.at[p, j],
            acc.at[plsc.Indices(dstl.at[p, j], ignored_value=SKIP)],
            ssem.at[p], add=True)


def _scatter_drain(rows, acc, dstl, ssem, p):
    for j in range(K_SUB):
        pltpu.make_async_copy(
            rows.at[p, j],
            acc.at[plsc.Indices(dstl.at[p, j], ignored_value=SKIP)],
            ssem.at[p]).wait()


def _make_edge_body(nc):
    def body(t_hbm, src_hbm, dst_hbm, out_hbm, srcb, dstb, srcm, dstl, rows,
             zbuf, acc, isem, gsem, ssem):
        c = lax.axis_index("c")
        s = lax.axis_index("s")
        slab0 = _zero_slab(s, zbuf, acc)
        row0 = s * TILE_ROWS
        for ci in range(nc):
            t_c = t_hbm.at[ci]
            plsc.subcore_barrier()
            for k0 in range(3):
                _idx_fire(src_hbm, srcb, isem, k0, row0 + k0 * K_SUB)
                _idx_fire(dst_hbm, dstb, isem, k0, row0 + k0 * K_SUB)

            @pl.loop(0, CHUNKS)
            def _(k):
                p = k & 3
                _idx_drain(src_hbm, srcb, isem, p)
                _idx_drain(dst_hbm, dstb, isem, p)

                @pl.when(k >= 4)
                def _():
                    _scatter_drain(rows, acc, dstl, ssem, p)

                for j in range(K_SUB):
                    _remap_both(c, srcb, dstb, srcm, dstl, p, j)
                _gather_fire(t_c, srcm, rows, gsem, p)

                @pl.when(k + 3 < CHUNKS)
                def _():
                    base3 = row0 + (k + 3) * K_SUB
                    pp3 = (k + 3) & 3
                    _idx_fire(src_hbm, srcb, isem, pp3, base3)
                    _idx_fire(dst_hbm, dstb, isem, pp3, base3)

                @pl.when(k >= 1)
                def _():
                    pm1 = (k - 1) & 3
                    _gather_drain(t_c, srcm, rows, gsem, pm1)
                    _scatter_fire(rows, acc, dstl, ssem, pm1)

            plast = (CHUNKS - 1) & 3
            _gather_drain(t_c, srcm, rows, gsem, plast)
            _scatter_fire(rows, acc, dstl, ssem, plast)
            for q in range(4):
                _scatter_drain(rows, acc, dstl, ssem, q)

            plsc.subcore_barrier()
            pltpu.sync_copy(acc.at[pl.ds(slab0, SLAB)],
                            out_hbm.at[ci, c, pl.ds(slab0, SLAB)])
            if ci + 1 < nc:
                _zero_slab(s, zbuf, acc)
    return body


def _ones_scatter_fire(onesb, acc, dstl, ssem, p):
    for j in range(K_SUB):
        pltpu.async_copy(
            onesb, acc.at[plsc.Indices(dstl.at[p, j], ignored_value=SKIP)],
            ssem.at[p], add=True)


def _ones_scatter_drain(onesb, acc, dstl, ssem, p):
    for j in range(K_SUB):
        pltpu.make_async_copy(
            onesb, acc.at[plsc.Indices(dstl.at[p, j], ignored_value=SKIP)],
            ssem.at[p]).wait()


def _degree_body(dst_hbm, out_hbm, dstb, dstl, onesb, zbuf, acc, isem, ssem):
    c = lax.axis_index("c")
    s = lax.axis_index("s")
    slab0 = _zero_slab(s, zbuf, acc, onesb=onesb)
    plsc.subcore_barrier()

    row0 = s * TILE_ROWS
    for k0 in range(3):
        _idx_fire(dst_hbm, dstb, isem, k0, row0 + k0 * K_SUB)

    @pl.loop(0, CHUNKS)
    def _(k):
        p = k & 3
        _idx_drain(dst_hbm, dstb, isem, p)

        @pl.when(k >= 4)
        def _():
            _ones_scatter_drain(onesb, acc, dstl, ssem, p)

        for j in range(K_SUB):
            _remap_dst(c, dstb, dstl, p, j)
        _ones_scatter_fire(onesb, acc, dstl, ssem, p)

        @pl.when(k + 3 < CHUNKS)
        def _():
            _idx_fire(dst_hbm, dstb, isem, (k + 3) & 3, row0 + (k + 3) * K_SUB)

    for q in range(4):
        _ones_scatter_drain(onesb, acc, dstl, ssem, q)

    plsc.subcore_barrier()
    pltpu.sync_copy(acc.at[pl.ds(slab0, SLAB)],
                    out_hbm.at[c, pl.ds(slab0, SLAB)])


@functools.cache
def _sc_mesh():
    return plsc.VectorSubcoreMesh(core_axis_name="c", subcore_axis_name="s",
                                  num_cores=2, num_subcores=16)


@functools.cache
def _edge_pass_fn(nc):
    return pl.kernel(
        _make_edge_body(nc),
        mesh=_sc_mesh(),
        compiler_params=pltpu.CompilerParams(use_tc_tiling_on_sc=False),
        out_type=jax.ShapeDtypeStruct((nc, 2, ACC_ROWS, 16), jnp.float32),
        scratch_types=[
            pltpu.VMEM((DEPTH, K_SUB, 128), jnp.int32),
            pltpu.VMEM((DEPTH, K_SUB, 128), jnp.int32),
            pltpu.VMEM((DEPTH, K_SUB, 128), jnp.int32),
            pltpu.VMEM((DEPTH, K_SUB, 128), jnp.int32),
            pltpu.VMEM((DEPTH, K_SUB, 128, 16), jnp.float32),
            pltpu.VMEM((128, 16), jnp.float32),
            pltpu.VMEM_SHARED((ACC_ROWS, 16), jnp.float32),
            pltpu.SemaphoreType.DMA((DEPTH,)),
            pltpu.SemaphoreType.DMA((DEPTH,)),
            pltpu.SemaphoreType.DMA((DEPTH,)),
        ],
    )


@functools.cache
def _degree_pass_fn():
    return pl.kernel(
        _degree_body,
        mesh=_sc_mesh(),
        compiler_params=pltpu.CompilerParams(use_tc_tiling_on_sc=False),
        out_type=jax.ShapeDtypeStruct((2, ACC_ROWS, 16), jnp.float32),
        scratch_types=[
            pltpu.VMEM((DEPTH, K_SUB, 128), jnp.int32),
            pltpu.VMEM((DEPTH, K_SUB, 128), jnp.int32),
            pltpu.VMEM((128, 16), jnp.float32),
            pltpu.VMEM((128, 16), jnp.float32),
            pltpu.VMEM_SHARED((ACC_ROWS, 16), jnp.float32),
            pltpu.SemaphoreType.DMA((DEPTH,)),
            pltpu.SemaphoreType.DMA((DEPTH,)),
        ],
    )


def _edge_pass(t3, src2d, dst2d):
    return _edge_pass_fn(t3.shape[0])(t3, src2d, dst2d)


def _degree_pass(dst2d):
    return _degree_pass_fn()(dst2d)


HALF_BLKS = HALF // ROWS_BLK


def _split_spec(prefix_map):
    def index_map(i):
        half = jnp.where(i < HALF_BLKS, 0, 1)
        row = jnp.where(i < HALF_BLKS, i, i - HALF_BLKS)
        return (*prefix_map, half, row, 0)

    return index_map


def _p_half_spec(ci):
    return pl.BlockSpec((1, 1, ROWS_BLK, 16), _split_spec((ci,)))


def _dinv_body(deg_ref, dinv_ref):
    dinv_ref[...] = lax.rsqrt(deg_ref[0] + 1.0)


def _dinv_call(degp):
    return pl.pallas_call(
        _dinv_body,
        grid=(GRID_N,),
        in_specs=[pl.BlockSpec((1, ROWS_BLK, 16), _split_spec(()))],
        out_specs=pl.BlockSpec((ROWS_BLK, 16), lambda i: (i, 0)),
        out_shape=jax.ShapeDtypeStruct((N_ACC, 16), jnp.float32),
    )(degp)


def _p1_body(x_ref, dinv_ref, w_ref, out_ref):
    res = jnp.dot(x_ref[...], w_ref[...], preferred_element_type=jnp.float32)
    out_ref[0] = res * dinv_ref[...]


def _p1_call(x_pad, dinv, w_pad):
    return pl.pallas_call(
        _p1_body,
        grid=(GRID_N,),
        in_specs=[
            pl.BlockSpec((ROWS_BLK, 16), lambda i: (i, 0)),
            pl.BlockSpec((ROWS_BLK, 16), lambda i: (i, 0)),
            pl.BlockSpec((16, 16), lambda i: (0, 0)),
        ],
        out_specs=pl.BlockSpec((1, ROWS_BLK, 16), lambda i: (0, i, 0)),
        out_shape=jax.ShapeDtypeStruct((1, N_ACC, 16), jnp.float32),
    )(x_pad, dinv, w_pad)


def _make_p_body(nc_in, nc_out, act):
    def body(*refs):
        t_ref = refs[0]
        p_refs = refs[1:1 + nc_in]
        dinv_ref = refs[1 + nc_in]
        b_ref = refs[2 + nc_in]
        w_ref = refs[3 + nc_in]
        out_ref = refs[4 + nc_in]
        dinv = dinv_ref[...]
        hs = []
        for ci in range(nc_in):
            agg = (p_refs[ci][0, 0] + t_ref[ci]) * dinv
            y = agg + b_ref[0, ci * 16:(ci + 1) * 16]
            hs.append(jnp.maximum(y, 0.0) if act else y)
        h = jnp.concatenate(hs, axis=1) if nc_in > 1 else hs[0]
        res = jnp.dot(h, w_ref[...], preferred_element_type=jnp.float32)
        t = res * dinv[:, 0:1]
        for cj in range(nc_out):
            out_ref[cj] = t[:, cj * 16:(cj + 1) * 16]
    return body


def _p_call(nc_in, nc_out, act, t3, p3, dinv, b_pad, w_pad):
    in_specs = (
        [pl.BlockSpec((nc_in, ROWS_BLK, 16), lambda i: (0, i, 0))]
        + [_p_half_spec(ci) for ci in range(nc_in)]
        + [pl.BlockSpec((ROWS_BLK, 16), lambda i: (i, 0)),
           pl.BlockSpec((1, nc_in * 16), lambda i: (0, 0)),
           pl.BlockSpec((nc_in * 16, nc_out * 16), lambda i: (0, 0))]
    )
    return pl.pallas_call(
        _make_p_body(nc_in, nc_out, act),
        grid=(GRID_N,),
        in_specs=in_specs,
        out_specs=pl.BlockSpec((nc_out, ROWS_BLK, 16), lambda i: (0, i, 0)),
        out_shape=jax.ShapeDtypeStruct((nc_out, N_ACC, 16), jnp.float32),
    )(t3, *([p3] * nc_in), dinv, b_pad, w_pad)


def _final_body(t_ref, p_ref, dinv_ref, b_ref, out_ref):
    y = (p_ref[0, 0] + t_ref[0]) * dinv_ref[...] + b_ref[0]
    out_ref[...] = y


def _final_call(t3, p3, dinv, b_pad):
    return pl.pallas_call(
        _final_body,
        grid=(GRID_N,),
        in_specs=[
            pl.BlockSpec((1, ROWS_BLK, 16), lambda i: (0, i, 0)),
            _p_half_spec(0),
            pl.BlockSpec((ROWS_BLK, 16), lambda i: (i, 0)),
            pl.BlockSpec((1, 16), lambda i: (0, 0)),
        ],
        out_specs=pl.BlockSpec((ROWS_BLK, 16), lambda i: (i, 0)),
        out_shape=jax.ShapeDtypeStruct((N_ACC, 16), jnp.float32),
    )(t3, p3, dinv, b_pad)


def _pad_w(w, rows, cols):
    return jnp.pad(w, ((0, rows - w.shape[0]), (0, cols - w.shape[1])))


def _pad_b(b, cols):
    return jnp.pad(b, (0, cols - b.shape[0])).reshape(1, cols)


def kernel(x, edge_index, W1, b1, W2, b2, W3, b3, W4, b4, W5, b5, W6, b6):
    n, _ = x.shape
    e = edge_index.shape[1]
    pad_e = E_PAD - e
    src2d = jnp.concatenate(
        [edge_index[0], jnp.zeros((pad_e,), jnp.int32)]).reshape(
            IDX_ROWS, 128)
    dst2d = jnp.concatenate(
        [edge_index[1], jnp.full((pad_e,), SKIP, jnp.int32)]).reshape(
            IDX_ROWS, 128)

    x_pad = jnp.pad(x, ((0, N_ACC - n), (0, 16 - x.shape[1])))

    deg = _degree_pass(dst2d)
    dinv = _dinv_call(deg)

    dims = [11, 16, 32, 50, 32, 16, 11]
    ncs = [-(-d // 16) for d in dims]
    ws = [W1, W2, W3, W4, W5, W6]
    bs = [b1, b2, b3, b4, b5, b6]
    acts = [True, True, False, True, True, False]

    t3 = _p1_call(x_pad, dinv, _pad_w(W1, 16, ncs[1] * 16))

    out_full = None
    for li in range(6):
        nc_in = ncs[li + 1]
        p3 = _edge_pass(t3, src2d, dst2d)
        if li < 5:
            nc_out = ncs[li + 2]
            t3 = _p_call(
                nc_in, nc_out, acts[li], t3, p3, dinv,
                _pad_b(bs[li], nc_in * 16),
                _pad_w(ws[li + 1], nc_in * 16, nc_out * 16))
        else:
            out_full = _final_call(t3, p3, dinv, _pad_b(bs[5], 16))
    return out_full[:n, :dims[6]]
